# baseline (speedup 1.0000x reference)
import functools

import jax
import jax.numpy as jnp
from jax import lax
from jax.experimental import pallas as pl
from jax.experimental.pallas import tpu as pltpu
from jax.experimental.pallas import tpu_sc as plsc

V = 1000
NC = 2
NS = 16
L = 16
NW = NC * NS
NR = 512
UNROLL = 8


def _lse_body(tab_ref, out_ref, copy_ref):
    x = tab_ref[...]
    m = jnp.max(x, axis=1, keepdims=True)
    s = jnp.sum(jnp.exp(x - m), axis=1, keepdims=True)
    out_ref[...] = m + jnp.log(s)
    copy_ref[...] = x


def _gather_body(ids_ref, tab_ref, out_ref):
    i = pl.program_id(0)

    def body(g, carry):
        for j in range(UNROLL):
            r = g * UNROLL + j
            idx = ids_ref[i * NR + r]
            out_ref[pl.ds(r, 1), :] = tab_ref[pl.ds(idx, 1), :]
        return carry

    lax.fori_loop(0, NR // UNROLL, body, 0)


@functools.lru_cache(maxsize=None)
def _make_tc_gather(B):
    grid_spec = pltpu.PrefetchScalarGridSpec(
        num_scalar_prefetch=1,
        grid=(B // NR,),
        in_specs=[pl.BlockSpec((V, V), lambda i, ids: (0, 0))],
        out_specs=pl.BlockSpec((NR, V), lambda i, ids: (i, 0)),
    )
    return pl.pallas_call(
        _gather_body,
        grid_spec=grid_spec,
        out_shape=jax.ShapeDtypeStruct((B, V), jnp.float32),
    )


@functools.lru_cache(maxsize=None)
def _make_sc_loss(B):
    SPW = B // NW
    GC = 80
    NG = SPW // GC
    mesh = plsc.VectorSubcoreMesh(core_axis_name="c", subcore_axis_name="s")

    @functools.partial(
        pl.kernel,
        mesh=mesh,
        compiler_params=pltpu.CompilerParams(use_tc_tiling_on_sc=False),
        out_type=jax.ShapeDtypeStruct((NW, L), jnp.float32),
        scratch_types=[
            pltpu.VMEM((SPW,), jnp.int32),
            pltpu.VMEM((SPW,), jnp.int32),
            pltpu.VMEM((SPW,), jnp.float32),
            pltpu.VMEM((SPW,), jnp.float32),
            pltpu.VMEM((L,), jnp.float32),
            pltpu.SemaphoreType.DMA,
        ],
    )
    def sc_loss(tabflat_hbm, ids_hbm, tgt_hbm, lse_hbm, part_hbm,
                ids_v, flat_v, lse_b, tv_b, acc_v, a):
        wid = lax.axis_index("s") * NC + lax.axis_index("c")
        base = wid * SPW
        pltpu.sync_copy(ids_hbm.at[pl.ds(base, SPW)], ids_v)
        pltpu.sync_copy(tgt_hbm.at[pl.ds(base, SPW)], flat_v)
        acc_v[...] = jnp.zeros((L,), jnp.float32)

        def flatten_idx(i, carry):
            sl = pl.ds(i * L, L)
            flat_v[sl] = flat_v[sl] + ids_v[sl] * V
            return carry

        lax.fori_loop(0, SPW // L, flatten_idx, 0)

        handles = []
        for gidx in range(NG):
            sl = pl.ds(gidx * GC, GC)
            hl = pltpu.make_async_copy(
                lse_hbm.at[ids_v.at[sl]], lse_b.at[sl], a)
            hl.start()
            ht = pltpu.make_async_copy(
                tabflat_hbm.at[flat_v.at[sl]], tv_b.at[sl], a)
            ht.start()
            handles.append(hl)
            handles.append(ht)
        for h in handles:
            h.wait()

        def accum(i, carry):
            sl = pl.ds(i * L, L)
            acc_v[...] = acc_v[...] + (lse_b[sl] - tv_b[sl])
            return carry

        lax.fori_loop(0, SPW // L, accum, 0)
        pltpu.sync_copy(acc_v, part_hbm.at[wid])

    return sc_loss


def kernel(input_ids, targets, token_embedding_table):
    B = input_ids.shape[0] * input_ids.shape[1]
    ids = input_ids.reshape(B).astype(jnp.int32)
    tgs = targets.reshape(B).astype(jnp.int32)
    lse, tabcopy = pl.pallas_call(
        _lse_body,
        out_shape=[
            jax.ShapeDtypeStruct((V, 1), jnp.float32),
            jax.ShapeDtypeStruct((V, V), jnp.float32),
        ],
    )(token_embedding_table)
    logits = _make_tc_gather(B)(ids, token_embedding_table)
    parts = _make_sc_loss(B)(
        tabcopy.reshape(V * V), ids, tgs, lse.reshape(V))
    loss = jnp.sum(parts) / B
    return logits, loss


# speedup vs baseline: 1.2782x; 1.2782x over previous
import functools

import jax
import jax.numpy as jnp
from jax import lax
from jax.experimental import pallas as pl
from jax.experimental.pallas import tpu as pltpu
from jax.experimental.pallas import tpu_sc as plsc

V = 1000
NC = 2
NS = 16
L = 16
NW = NC * NS
LSE_T = 7
LSE_L = 104


def _prep_body(tab_ref, out_ref):
    x = tab_ref[...]
    m = jnp.max(x, axis=1, keepdims=True)
    s = jnp.sum(jnp.exp(x - m), axis=1, keepdims=True)
    lse = m + jnp.log(s)
    for k in range(7):
        out_ref[:, k, :] = x[:, k * 128:(k + 1) * 128]
    out_ref[:, 7, 0:104] = x[:, 896:1000]
    out_ref[:, 7, 104:105] = lse
    out_ref[:, 7, 105:128] = jnp.zeros((V, 23), jnp.float32)


@functools.lru_cache(maxsize=None)
def _make_sc_gather(B):
    SPW = B // NW
    CH = 16
    NCH = SPW // CH
    mesh = plsc.VectorSubcoreMesh(core_axis_name="c", subcore_axis_name="s")

    @functools.partial(
        pl.kernel,
        mesh=mesh,
        compiler_params=pltpu.CompilerParams(
            use_tc_tiling_on_sc=True, disable_bounds_checks=True,
            needs_layout_passes=False),
        out_type=[
            jax.ShapeDtypeStruct((B, V), jnp.float32),
            jax.ShapeDtypeStruct((NW * 8, 128), jnp.float32),
        ],
        scratch_types=[
            pltpu.VMEM((SPW,), jnp.int32),
            pltpu.VMEM((SPW,), jnp.int32),
            pltpu.VMEM((CH, 8, 128), jnp.float32),
            pltpu.VMEM((CH, 8, 128), jnp.float32),
            pltpu.VMEM((CH, 8, 128), jnp.float32),
            pltpu.VMEM((CH, 8, 128), jnp.float32),
            pltpu.VMEM((8, 128), jnp.float32),
            pltpu.VMEM((L,), jnp.float32),
            pltpu.SemaphoreType.DMA,
            pltpu.SemaphoreType.DMA,
            pltpu.SemaphoreType.DMA,
            pltpu.SemaphoreType.DMA,
            pltpu.SemaphoreType.DMA,
            pltpu.SemaphoreType.DMA,
            pltpu.SemaphoreType.DMA,
            pltpu.SemaphoreType.DMA,
        ],
    )
    def sc_gather(tab3_hbm, ids_hbm, tgt_hbm, out_hbm, part_hbm,
                  ids_v, tgt_v, rows0, rows1, rows2, rows3, stage, acc_v,
                  g0, g1, g2, g3, w0, w1, w2, w3):
        wid = lax.axis_index("s") * NC + lax.axis_index("c")
        base = wid * SPW
        pltpu.sync_copy(ids_hbm.at[pl.ds(base, SPW)], ids_v)
        pltpu.sync_copy(tgt_hbm.at[pl.ds(base, SPW)], tgt_v)
        acc_v[...] = jnp.zeros((L,), jnp.float32)

        def start_gather(c, rows_b, gsem):
            h = pltpu.make_async_copy(
                tab3_hbm.at[ids_v.at[pl.ds(c * CH, CH)]], rows_b, gsem)
            h.start()
            return h

        def write_chunk(c, rows_b, wsem):
            r0 = base + c * CH
            zero = wid * 0
            hs = []
            for t in range(8):
                h = pltpu.make_async_copy(
                    rows_b.at[:, t, :],
                    out_hbm.at[pl.ds(r0, CH), pl.ds(zero + t * 128, 128)],
                    wsem)
                h.start()
                hs.append(h)
            return hs

        def loss_chunk(c, rows_b):
            for j in range(CH // L):
                rloc = lax.iota(jnp.int32, 16) + j * L
                t16 = tgt_v[pl.ds(c * CH + j * L, L)]
                lse16 = plsc.load_gather(
                    rows_b, [rloc, jnp.full((L,), LSE_T, jnp.int32),
                             jnp.full((L,), LSE_L, jnp.int32)])
                tv16 = plsc.load_gather(
                    rows_b, [rloc, lax.shift_right_logical(t16, 7),
                             lax.bitwise_and(t16, 127)])
                acc_v[...] = acc_v[...] + (lse16 - tv16)

        ROWS = (rows0, rows1, rows2, rows3)
        GS = (g0, g1, g2, g3)
        WS = (w0, w1, w2, w3)

        def drain_writes(buf, wsem):
            for _ in range(8):
                pltpu.make_async_copy(
                    buf.at[:, 0, :],
                    out_hbm.at[pl.ds(base, CH), pl.ds(0, 128)], wsem).wait()

        start_gather(0, rows0, g0)
        start_gather(1, rows1, g1)

        def body(g, carry):
            for b in range(4):
                c = 4 * g + b
                nb = (b + 2) % 4
                pltpu.make_async_copy(
                    tab3_hbm.at[ids_v.at[pl.ds(c * CH, CH)]], ROWS[b],
                    GS[b]).wait()
                write_chunk(c, ROWS[b], WS[b])
                loss_chunk(c, ROWS[b])

                @pl.when(c + 2 < NCH)
                def _start_next():
                    @pl.when(c - 2 >= 0)
                    def _drain_prev():
                        drain_writes(ROWS[nb], WS[nb])
                    start_gather(c + 2, ROWS[nb], GS[nb])
            return carry

        lax.fori_loop(0, NCH // 4, body, 0)
        for b in range(4):
            drain_writes(ROWS[b], WS[b])
        stage[0, pl.ds(0, L)] = acc_v[...]
        pltpu.sync_copy(stage, part_hbm.at[pl.ds(wid * 8, 8), :])

    return sc_gather


def kernel(input_ids, targets, token_embedding_table):
    B = input_ids.shape[0] * input_ids.shape[1]
    ids = input_ids.reshape(B).astype(jnp.int32)
    tgs = targets.reshape(B).astype(jnp.int32)
    tab3 = pl.pallas_call(
        _prep_body,
        out_shape=jax.ShapeDtypeStruct((V, 8, 128), jnp.float32),
    )(token_embedding_table)
    logits, parts = _make_sc_gather(B)(tab3, ids, tgs)
    loss = jnp.sum(parts[0::8, 0:L]) / B
    return logits, loss


# speedup vs baseline: 1.2787x; 1.0004x over previous
import functools

import jax
import jax.numpy as jnp
from jax import lax
from jax.experimental import pallas as pl
from jax.experimental.pallas import tpu as pltpu
from jax.experimental.pallas import tpu_sc as plsc

V = 1000
NC = 2
NS = 16
L = 16
NW = NC * NS
LSE_T = 7
LSE_L = 104


def _prep_body(tab_ref, out_ref):
    x = tab_ref[...]
    m = jnp.max(x, axis=1, keepdims=True)
    s = jnp.sum(jnp.exp(x - m), axis=1, keepdims=True)
    lse = m + jnp.log(s)
    for k in range(7):
        out_ref[:, k, :] = x[:, k * 128:(k + 1) * 128]
    out_ref[:, 7, 0:104] = x[:, 896:1000]
    out_ref[:, 7, 104:105] = lse
    out_ref[:, 7, 105:128] = jnp.zeros((V, 23), jnp.float32)


@functools.lru_cache(maxsize=None)
def _make_sc_gather(B):
    SPW = B // NW
    CH = 40
    NCH = SPW // CH
    mesh = plsc.VectorSubcoreMesh(core_axis_name="c", subcore_axis_name="s")

    @functools.partial(
        pl.kernel,
        mesh=mesh,
        compiler_params=pltpu.CompilerParams(
            use_tc_tiling_on_sc=True, disable_bounds_checks=True,
            needs_layout_passes=False),
        out_type=[
            jax.ShapeDtypeStruct((B, V), jnp.float32),
            jax.ShapeDtypeStruct((NW * 8, 128), jnp.float32),
        ],
        scratch_types=[
            pltpu.VMEM((SPW,), jnp.int32),
            pltpu.VMEM((SPW,), jnp.int32),
            pltpu.VMEM((CH, 8, 128), jnp.float32),
            pltpu.VMEM((CH, 8, 128), jnp.float32),
            pltpu.VMEM((8, 128), jnp.float32),
            pltpu.VMEM((L,), jnp.float32),
            pltpu.SemaphoreType.DMA,
            pltpu.SemaphoreType.DMA,
            pltpu.SemaphoreType.DMA,
            pltpu.SemaphoreType.DMA,
            pltpu.SemaphoreType.DMA,
        ],
    )
    def sc_gather(tab3_hbm, ids_hbm, tgt_hbm, out_hbm, part_hbm,
                  ids_v, tgt_v, rows0, rows1, stage, acc_v,
                  g0, g1, w0, w1, ps):
        wid = lax.axis_index("s") * NC + lax.axis_index("c")
        base = wid * SPW
        pltpu.sync_copy(ids_hbm.at[pl.ds(base, SPW)], ids_v)
        pltpu.sync_copy(tgt_hbm.at[pl.ds(base, SPW)], tgt_v)
        acc_v[...] = jnp.zeros((L,), jnp.float32)

        def start_gather(c, rows_b, gsem):
            h = pltpu.make_async_copy(
                tab3_hbm.at[ids_v.at[pl.ds(c * CH, CH)]], rows_b, gsem)
            h.start()
            return h

        def write_chunk(c, rows_b, wsem):
            r0 = base + c * CH
            zero = wid * 0
            hs = []
            for t in range(8):
                h = pltpu.make_async_copy(
                    rows_b.at[:, t, :],
                    out_hbm.at[pl.ds(r0, CH), pl.ds(zero + t * 128, 128)],
                    wsem)
                h.start()
                hs.append(h)
            return hs

        def loss_chunk(c, rows_b):
            for j in range((CH + L - 1) // L):
                rloc = lax.iota(jnp.int32, 16) + j * L
                valid = rloc < CH
                t16 = tgt_v[pl.ds(c * CH + j * L, L)]
                lse16 = plsc.load_gather(
                    rows_b, [rloc, jnp.full((L,), LSE_T, jnp.int32),
                             jnp.full((L,), LSE_L, jnp.int32)], mask=valid)
                tv16 = plsc.load_gather(
                    rows_b, [rloc, lax.shift_right_logical(t16, 7),
                             lax.bitwise_and(t16, 127)], mask=valid)
                nll = lse16 - tv16
                if CH % L:
                    nll = jnp.where(valid, nll, 0.0)
                acc_v[...] = acc_v[...] + nll

        def body(g, carry):
            c0 = 2 * g
            c1 = 2 * g + 1
            hg0 = start_gather(c0, rows0, g0)
            hg1 = start_gather(c1, rows1, g1)
            hg0.wait()
            hw0 = write_chunk(c0, rows0, w0)
            loss_chunk(c0, rows0)
            hg1.wait()
            hw1 = write_chunk(c1, rows1, w1)
            loss_chunk(c1, rows1)
            for h in hw0:
                h.wait()
            for h in hw1:
                h.wait()
            return carry

        lax.fori_loop(0, NCH // 2, body, 0)
        stage[0, pl.ds(0, L)] = acc_v[...]
        pltpu.sync_copy(stage, part_hbm.at[pl.ds(wid * 8, 8), :])

    return sc_gather


def kernel(input_ids, targets, token_embedding_table):
    B = input_ids.shape[0] * input_ids.shape[1]
    ids = input_ids.reshape(B).astype(jnp.int32)
    tgs = targets.reshape(B).astype(jnp.int32)
    tab3 = pl.pallas_call(
        _prep_body,
        out_shape=jax.ShapeDtypeStruct((V, 8, 128), jnp.float32),
    )(token_embedding_table)
    logits, parts = _make_sc_gather(B)(tab3, ids, tgs)
    loss = jnp.sum(parts[0::8, 0:L]) / B
    return logits, loss
